# Initial kernel scaffold; baseline (speedup 1.0000x reference)
import jax
import jax.numpy as jnp
from jax.experimental import pallas as pl


def kernel(drug_name, adj_tail, adj_relation, drug_table, rela_table, ent_table, W_lin, b_lin, gamma, beta):
    raise NotImplementedError("write your pallas kernel here")


# speedup vs baseline: 1.5464x; 1.5464x over previous
import functools

import jax
import jax.numpy as jnp
from jax import lax
from jax.experimental import pallas as pl
from jax.experimental.pallas import tpu as pltpu
from jax.experimental.pallas import tpu_sc as plsc

ND = 572
K = 64
D = 128
NR = 200
NP = 576
NB = NP // 8
NW = 32
L = 16


def _scores_matmul(drug_pad, rela_t):
    def body(a_ref, b_ref, o_ref):
        o_ref[...] = jnp.dot(a_ref[...], b_ref[...],
                             preferred_element_type=jnp.float32)
    return pl.pallas_call(
        body, out_shape=jax.ShapeDtypeStruct((NP, 2 * D), jnp.float32)
    )(drug_pad, rela_t)


def _head(x, w, b, gamma, beta):
    def body(x_ref, w_ref, b_ref, g_ref, bt_ref, o_ref):
        h = jnp.dot(x_ref[...], w_ref[...],
                    preferred_element_type=jnp.float32) + b_ref[...]
        h = jnp.maximum(h, 0.0)
        rows = lax.broadcasted_iota(jnp.int32, (NP, D), 0)
        mask = rows < ND
        hm = jnp.where(mask, h, 0.0)
        mean = jnp.sum(hm, axis=0, keepdims=True) * (1.0 / ND)
        dlt = jnp.where(mask, h - mean, 0.0)
        var = jnp.sum(dlt * dlt, axis=0, keepdims=True) * (1.0 / ND)
        o_ref[...] = (g_ref[...] * (h - mean) * lax.rsqrt(var + 1e-5)
                      + bt_ref[...])
    return pl.pallas_call(
        body, out_shape=jax.ShapeDtypeStruct((NP, D), jnp.float32)
    )(x, w, b, gamma, beta)


@functools.cache
def _make_sc_attend():
    mesh = plsc.VectorSubcoreMesh(core_axis_name="c", subcore_axis_name="s")

    @functools.partial(
        pl.kernel,
        out_type=jax.ShapeDtypeStruct((NP, 2 * D), jnp.float32),
        mesh=mesh,
        scratch_types=[
            pltpu.VMEM((8,), jnp.int32),
            pltpu.VMEM((8, K), jnp.int32),
            pltpu.VMEM((8, K), jnp.int32),
            pltpu.VMEM((8, 2 * D), jnp.float32),
            pltpu.VMEM((8, D), jnp.float32),
            pltpu.VMEM((K, D), jnp.float32),
            pltpu.VMEM((K + L,), jnp.float32),
            pltpu.VMEM((8, 2 * D), jnp.float32),
            pltpu.SemaphoreType.DMA,
        ],
        compiler_params=pltpu.CompilerParams(use_tc_tiling_on_sc=False,
                                             needs_layout_passes=False),
    )
    def sc_attend(s_hbm, name_hbm, tail_hbm, rel_hbm, drug_hbm, ent_hbm,
                  out_hbm, name_v, tail_v, rel_v, s_rows, drug_rows,
                  ent_buf, w_ref, out_buf, sem):
        wid = lax.axis_index("s") * 2 + lax.axis_index("c")
        for b in range(3):
            blk = wid + NW * b

            @pl.when(blk < NB)
            def _():
                base = blk * 8
                pltpu.sync_copy(name_hbm.at[pl.ds(base, 8)], name_v)
                pltpu.sync_copy(tail_hbm.at[pl.ds(base, 8)], tail_v)
                pltpu.sync_copy(rel_hbm.at[pl.ds(base, 8)], rel_v)
                pltpu.async_copy(s_hbm.at[name_v], s_rows, sem).wait()
                pltpu.async_copy(drug_hbm.at[name_v], drug_rows, sem).wait()
                for r in range(8):
                    pltpu.async_copy(ent_hbm.at[tail_v.at[r]], ent_buf,
                                     sem).wait()
                    row_idx = jnp.full((L,), r, jnp.int32)
                    svecs = []
                    for c in range(4):
                        col = rel_v[r, pl.ds(c * L, L)]
                        svecs.append(plsc.load_gather(s_rows, [row_idx, col]))
                    m = jnp.max(jnp.maximum(jnp.maximum(svecs[0], svecs[1]),
                                            jnp.maximum(svecs[2], svecs[3])))
                    evecs = [jnp.exp(sv - m) for sv in svecs]
                    tot = jnp.sum(evecs[0] + evecs[1] + evecs[2] + evecs[3])
                    inv = 1.0 / jnp.broadcast_to(tot, (L,))
                    for c in range(4):
                        w_ref[pl.ds(c * L, L)] = evecs[c] * inv

                    def kbody(k, acc):
                        wk = w_ref[pl.ds(k, L)][0]
                        return tuple(acc[dc] + wk * ent_buf[k, pl.ds(dc * L, L)]
                                     for dc in range(8))
                    acc = lax.fori_loop(
                        0, K, kbody,
                        tuple(jnp.zeros((L,), jnp.float32) for _ in range(8)))
                    for dc in range(8):
                        out_buf[r, pl.ds(dc * L, L)] = acc[dc]
                        out_buf[r, pl.ds(D + dc * L, L)] = \
                            drug_rows[r, pl.ds(dc * L, L)]
                pltpu.sync_copy(out_buf, out_hbm.at[pl.ds(base, 8)])

    return sc_attend


def kernel(drug_name, adj_tail, adj_relation, drug_table, rela_table,
           ent_table, W_lin, b_lin, gamma, beta):
    name = jnp.pad(drug_name.astype(jnp.int32), (0, NP - ND))
    tail = jnp.pad(adj_tail.astype(jnp.int32), ((0, NP - ND), (0, 0)))
    rel = jnp.pad(adj_relation.astype(jnp.int32), ((0, NP - ND), (0, 0)))
    drug_pad = jnp.pad(drug_table, ((0, NP - ND), (0, 0)))
    rela_t = jnp.pad(rela_table, ((0, 2 * D - NR), (0, 0))).T

    s = _scores_matmul(drug_pad, rela_t)
    dq = _make_sc_attend()(s, name, tail, rel, drug_table, ent_table)
    out = _head(dq, W_lin, b_lin.reshape(1, D), gamma.reshape(1, D),
                beta.reshape(1, D))
    return out[:ND]


# speedup vs baseline: 2.1186x; 1.3700x over previous
import functools

import jax
import jax.numpy as jnp
from jax import lax
from jax.experimental import pallas as pl
from jax.experimental.pallas import tpu as pltpu
from jax.experimental.pallas import tpu_sc as plsc

ND = 572
K = 64
D = 128
NR = 200
NP = 576
NW = 32
RW = NP // NW
L = 16


def _scores_matmul(drug_pad, rela_t):
    def body(a_ref, b_ref, o_ref):
        o_ref[...] = jnp.dot(a_ref[...], b_ref[...],
                             preferred_element_type=jnp.float32,
                             precision=lax.Precision.HIGHEST)
    return pl.pallas_call(
        body, out_shape=jax.ShapeDtypeStruct((NP, 2 * D), jnp.float32)
    )(drug_pad, rela_t)


def _head(x, w, b, gamma, beta):
    def body(x_ref, w_ref, b_ref, g_ref, bt_ref, o_ref):
        h = jnp.dot(x_ref[...], w_ref[...],
                    preferred_element_type=jnp.float32,
                    precision=lax.Precision.HIGHEST) + b_ref[...]
        h = jnp.maximum(h, 0.0)
        rows = lax.broadcasted_iota(jnp.int32, (NP, D), 0)
        mask = rows < ND
        hm = jnp.where(mask, h, 0.0)
        mean = jnp.sum(hm, axis=0, keepdims=True) * (1.0 / ND)
        dlt = jnp.where(mask, h - mean, 0.0)
        var = jnp.sum(dlt * dlt, axis=0, keepdims=True) * (1.0 / ND)
        o_ref[...] = (g_ref[...] * (h - mean) * lax.rsqrt(var + 1e-5)
                      + bt_ref[...])
    return pl.pallas_call(
        body, out_shape=jax.ShapeDtypeStruct((NP, D), jnp.float32)
    )(x, w, b, gamma, beta)


@functools.cache
def _make_sc_attend():
    mesh = plsc.VectorSubcoreMesh(core_axis_name="c", subcore_axis_name="s")

    @functools.partial(
        pl.kernel,
        out_type=jax.ShapeDtypeStruct((NP, 2 * D), jnp.float32),
        mesh=mesh,
        scratch_types=[
            pltpu.VMEM((RW,), jnp.int32),
            pltpu.VMEM((RW, K), jnp.int32),
            pltpu.VMEM((RW, K), jnp.int32),
            pltpu.VMEM((RW, 2 * D), jnp.float32),
            pltpu.VMEM((RW, D), jnp.float32),
            pltpu.VMEM((K, D), jnp.float32),
            pltpu.VMEM((K, D), jnp.float32),
            pltpu.VMEM((K + L,), jnp.float32),
            pltpu.VMEM((RW, 2 * D), jnp.float32),
            pltpu.SemaphoreType.DMA,
            pltpu.SemaphoreType.DMA,
            pltpu.SemaphoreType.DMA,
        ],
        compiler_params=pltpu.CompilerParams(use_tc_tiling_on_sc=False,
                                             needs_layout_passes=False),
    )
    def sc_attend(s_hbm, name_hbm, tail_hbm, rel_hbm, drug_hbm, ent_hbm,
                  out_hbm, name_v, tail_v, rel_v, s_rows, drug_rows,
                  ent_buf0, ent_buf1, w_ref, out_buf, sem, esem0, esem1):
        wid = lax.axis_index("s") * 2 + lax.axis_index("c")
        ent_bufs = (ent_buf0, ent_buf1)
        esems = (esem0, esem1)
        base = wid * RW
        pltpu.sync_copy(tail_hbm.at[pl.ds(base, RW)], tail_v)
        descs = [pltpu.async_copy(ent_hbm.at[tail_v.at[0]],
                                  ent_bufs[0], esems[0])]
        pltpu.sync_copy(name_hbm.at[wid], name_v)
        pltpu.sync_copy(rel_hbm.at[pl.ds(base, RW)], rel_v)
        pltpu.async_copy(s_hbm.at[name_v], s_rows, sem).wait()
        pltpu.async_copy(drug_hbm.at[name_v], drug_rows, sem).wait()
        for r in range(RW):
            if r < RW - 1:
                descs.append(pltpu.async_copy(
                    ent_hbm.at[tail_v.at[r + 1]],
                    ent_bufs[(r + 1) % 2], esems[(r + 1) % 2]))
            descs[r].wait()
            ent_buf = ent_bufs[r % 2]
            row_idx = jnp.full((L,), r, jnp.int32)
            svecs = []
            for c in range(4):
                col = rel_v[r, pl.ds(c * L, L)]
                svecs.append(plsc.load_gather(s_rows, [row_idx, col]))
            m = jnp.max(jnp.maximum(jnp.maximum(svecs[0], svecs[1]),
                                    jnp.maximum(svecs[2], svecs[3])))
            evecs = [jnp.exp(sv - m) for sv in svecs]
            tot = jnp.sum(evecs[0] + evecs[1] + evecs[2] + evecs[3])
            inv = 1.0 / jnp.broadcast_to(tot, (L,))
            for c in range(4):
                w_ref[pl.ds(c * L, L)] = evecs[c] * inv

            def kbody(k, acc):
                wk = w_ref[pl.ds(k, L)][0]
                return tuple(acc[dc] + wk * ent_buf[k, pl.ds(dc * L, L)]
                             for dc in range(8))
            acc = lax.fori_loop(
                0, K, kbody,
                tuple(jnp.zeros((L,), jnp.float32) for _ in range(8)),
                unroll=4)
            for dc in range(8):
                out_buf[r, pl.ds(dc * L, L)] = acc[dc]
                out_buf[r, pl.ds(D + dc * L, L)] = \
                    drug_rows[r, pl.ds(dc * L, L)]
        pltpu.sync_copy(out_buf, out_hbm.at[pl.ds(base, RW)])

    return sc_attend


def kernel(drug_name, adj_tail, adj_relation, drug_table, rela_table,
           ent_table, W_lin, b_lin, gamma, beta):
    name = jnp.pad(drug_name.astype(jnp.int32), (0, NP - ND)).reshape(NW, RW)
    tail = jnp.pad(adj_tail.astype(jnp.int32), ((0, NP - ND), (0, 0)))
    rel = jnp.pad(adj_relation.astype(jnp.int32), ((0, NP - ND), (0, 0)))
    drug_pad = jnp.pad(drug_table, ((0, NP - ND), (0, 0)))
    rela_t = jnp.pad(rela_table, ((0, 2 * D - NR), (0, 0))).T

    s = _scores_matmul(drug_pad, rela_t)
    dq = _make_sc_attend()(s, name, tail, rel, drug_table, ent_table)
    out = _head(dq, W_lin, b_lin.reshape(1, D), gamma.reshape(1, D),
                beta.reshape(1, D))
    return out[:ND]


# speedup vs baseline: 2.9949x; 1.4137x over previous
import functools

import jax
import jax.numpy as jnp
from jax import lax
from jax.experimental import pallas as pl
from jax.experimental.pallas import tpu as pltpu
from jax.experimental.pallas import tpu_sc as plsc

ND = 572
K = 64
D = 128
NR = 200
NW = 32
RW = 18
L = 16
NBUF = 4


def _scores_matmul(drug_table, rela_t):
    def body(a_ref, b_ref, o_ref):
        o_ref[...] = jnp.dot(a_ref[...], b_ref[...],
                             preferred_element_type=jnp.float32,
                             precision=lax.Precision.HIGHEST)
    return pl.pallas_call(
        body, out_shape=jax.ShapeDtypeStruct((ND, 2 * D), jnp.float32)
    )(drug_table, rela_t)


def _head(x, w, b, gamma, beta):
    def body(x_ref, w_ref, b_ref, g_ref, bt_ref, o_ref):
        h = jnp.dot(x_ref[...], w_ref[...],
                    preferred_element_type=jnp.float32,
                    precision=lax.Precision.HIGHEST) + b_ref[...]
        h = jnp.maximum(h, 0.0)
        mean = jnp.sum(h, axis=0, keepdims=True) * (1.0 / ND)
        dlt = h - mean
        var = jnp.sum(dlt * dlt, axis=0, keepdims=True) * (1.0 / ND)
        o_ref[...] = (g_ref[...] * dlt * lax.rsqrt(var + 1e-5) + bt_ref[...])
    return pl.pallas_call(
        body, out_shape=jax.ShapeDtypeStruct((ND, D), jnp.float32)
    )(x, w, b, gamma, beta)


@functools.cache
def _make_sc_attend():
    mesh = plsc.VectorSubcoreMesh(core_axis_name="c", subcore_axis_name="s")

    @functools.partial(
        pl.kernel,
        out_type=jax.ShapeDtypeStruct((ND, 2 * D), jnp.float32),
        mesh=mesh,
        scratch_types=[
            pltpu.VMEM((RW,), jnp.int32),
            pltpu.VMEM((RW, K), jnp.int32),
            pltpu.VMEM((RW, K), jnp.int32),
            pltpu.VMEM((RW, 2 * D), jnp.float32),
            pltpu.VMEM((RW, D), jnp.float32),
            [pltpu.VMEM((K, D), jnp.float32) for _ in range(NBUF)],
            pltpu.VMEM((K + L,), jnp.float32),
            pltpu.VMEM((RW, 2 * D), jnp.float32),
            pltpu.SemaphoreType.DMA,
            [pltpu.SemaphoreType.DMA for _ in range(NBUF)],
        ],
        compiler_params=pltpu.CompilerParams(use_tc_tiling_on_sc=False,
                                             needs_layout_passes=False),
    )
    def sc_attend(s_hbm, name_hbm, tail_hbm, rel_hbm, drug_hbm, ent_hbm,
                  out_hbm, name_v, tail_v, rel_v, s_rows, drug_rows,
                  ent_bufs, w_ref, out_buf, sem, esems):
        wid = lax.axis_index("s") * 2 + lax.axis_index("c")
        base = jnp.minimum(wid * RW, ND - RW)
        pltpu.sync_copy(tail_hbm.at[pl.ds(base, RW)], tail_v)
        for i in range(NBUF - 1):
            pltpu.async_copy(ent_hbm.at[tail_v.at[i]], ent_bufs[i], esems[i])
        pltpu.sync_copy(name_hbm.at[wid], name_v)
        pltpu.sync_copy(rel_hbm.at[pl.ds(base, RW)], rel_v)
        pltpu.async_copy(s_hbm.at[name_v], s_rows, sem).wait()
        pltpu.async_copy(drug_hbm.at[name_v], drug_rows, sem).wait()

        def row_body(r, slot):
            ent_buf, esem = ent_bufs[slot], esems[slot]
            issue_slot = (slot - 1) % NBUF
            nxt = r + NBUF - 1

            @pl.when(nxt < RW)
            def _():
                pltpu.async_copy(ent_hbm.at[tail_v.at[nxt]],
                                 ent_bufs[issue_slot], esems[issue_slot])

            row_idx = jnp.broadcast_to(r, (L,)).astype(jnp.int32)
            svecs = []
            for c in range(4):
                col = rel_v[r, pl.ds(c * L, L)]
                svecs.append(plsc.load_gather(s_rows, [row_idx, col]))
            m = jnp.max(jnp.maximum(jnp.maximum(svecs[0], svecs[1]),
                                    jnp.maximum(svecs[2], svecs[3])))
            evecs = [jnp.exp(sv - m) for sv in svecs]
            tot = jnp.sum(evecs[0] + evecs[1] + evecs[2] + evecs[3])
            inv = 1.0 / jnp.broadcast_to(tot, (L,))
            for c in range(4):
                w_ref[pl.ds(c * L, L)] = evecs[c] * inv

            pltpu.make_async_copy(ent_hbm.at[tail_v.at[r]], ent_buf,
                                  esem).wait()

            zeros = tuple(jnp.zeros((L,), jnp.float32) for _ in range(8))

            @plsc.parallel_loop(0, K, 1, unroll=4, carry=zeros)
            def acc(k, a):
                wk = w_ref[pl.ds(k, L)][0]
                return tuple(a[dc] + wk * ent_buf[k, pl.ds(dc * L, L)]
                             for dc in range(8))

            for dc in range(8):
                out_buf[r, pl.ds(dc * L, L)] = acc[dc]
                out_buf[r, pl.ds(D + dc * L, L)] = \
                    drug_rows[r, pl.ds(dc * L, L)]

        def group_body(p, carry):
            for j in range(NBUF):
                row_body(p * NBUF + j, j)
            return carry

        lax.fori_loop(0, RW // NBUF, group_body, 0)
        for j in range(RW - RW % NBUF, RW):
            row_body(jnp.int32(j), j % NBUF)
        pltpu.sync_copy(out_buf, out_hbm.at[pl.ds(base, RW)])

    return sc_attend


def kernel(drug_name, adj_tail, adj_relation, drug_table, rela_table,
           ent_table, W_lin, b_lin, gamma, beta):
    name = drug_name.astype(jnp.int32)
    tail = adj_tail.astype(jnp.int32)
    rel = adj_relation.astype(jnp.int32)
    bases = jnp.minimum(jnp.arange(NW, dtype=jnp.int32) * RW, ND - RW)
    name2d = name[bases[:, None] + jnp.arange(RW, dtype=jnp.int32)[None, :]]
    rela_t = jnp.pad(rela_table, ((0, 2 * D - NR), (0, 0))).T

    s = _scores_matmul(drug_table, rela_t)
    dq = _make_sc_attend()(s, name2d, tail, rel, drug_table, ent_table)
    return _head(dq, W_lin, b_lin.reshape(1, D), gamma.reshape(1, D),
                 beta.reshape(1, D))
